# baseline (speedup 1.0000x reference)
import functools

import jax
import jax.numpy as jnp
from jax import lax
from jax.experimental import pallas as pl
from jax.experimental.pallas import tpu as pltpu
from jax.experimental.pallas import tpu_sc as plsc

B = 4
T = 8192
E = 1024

_info = plsc.get_sparse_core_info()
_NC = _info.num_cores
_NS = _info.num_subcores
_NW = _NC * _NS
_ROWS_PER_W = T // _NW
_NBUF = 2
_SIZES = (64, 56, 64, 56, 16)
_OFFS = tuple(sum(_SIZES[:i]) for i in range(len(_SIZES)))
_BOFF = (0, 64)
_NCHUNK = len(_SIZES)

_mesh = plsc.VectorSubcoreMesh(core_axis_name="c", subcore_axis_name="s")


@functools.partial(
    pl.kernel,
    mesh=_mesh,
    out_type=jax.ShapeDtypeStruct((B, T, E), jnp.float32),
    scratch_types=[
        pltpu.VMEM((120, E), jnp.float32),
        pltpu.SemaphoreType.DMA((_NBUF,)),
        pltpu.SemaphoreType.DMA((_NBUF,)),
    ],
)
def _broadcast_rows(table_hbm, out_hbm, buf, rsem, wsem):
    wid = lax.axis_index("s") * _NC + lax.axis_index("c")
    base = wid * _ROWS_PER_W

    def read_copy(c):
        k = c % _NBUF
        return pltpu.make_async_copy(
            table_hbm.at[pl.ds(base + _OFFS[c], _SIZES[c])],
            buf.at[pl.ds(_BOFF[k], _SIZES[c])],
            rsem.at[k],
        )

    def write_copy(c, b):
        k = c % _NBUF
        return pltpu.make_async_copy(
            buf.at[pl.ds(_BOFF[k], _SIZES[c])],
            out_hbm.at[b, pl.ds(base + _OFFS[c], _SIZES[c])],
            wsem.at[k],
        )

    for c in range(min(_NBUF - 1, _NCHUNK)):
        read_copy(c).start()
    for c in range(_NCHUNK):
        read_copy(c).wait()
        for b in range(B):
            write_copy(c, b).start()
        nxt = c + _NBUF - 1
        if nxt < _NCHUNK:
            if c >= 1:
                for b in range(B):
                    write_copy(c - 1, b).wait()
            read_copy(nxt).start()
    for c in range(max(0, _NCHUNK - _NBUF), _NCHUNK):
        for b in range(B):
            write_copy(c, b).wait()


def kernel(x, table):
    del x
    return _broadcast_rows(table)


# speedup vs baseline: 1.0052x; 1.0052x over previous
import functools

import jax
import jax.numpy as jnp
from jax import lax
from jax.experimental import pallas as pl
from jax.experimental.pallas import tpu as pltpu
from jax.experimental.pallas import tpu_sc as plsc

B = 4
T = 8192
E = 1024

_info = plsc.get_sparse_core_info()
_NC = _info.num_cores
_NS = _info.num_subcores
_NW = _NC * _NS
_ROWS_PER_W = T // _NW
_NBUF = 2
_SIZES = (64, 56, 64, 56, 16)
_OFFS = tuple(sum(_SIZES[:i]) for i in range(len(_SIZES)))
_BOFF = (0, 64)
_NCHUNK = len(_SIZES)

_mesh = plsc.VectorSubcoreMesh(core_axis_name="c", subcore_axis_name="s")


@functools.partial(
    pl.kernel,
    mesh=_mesh,
    out_type=jax.ShapeDtypeStruct((B, T, E), jnp.float32),
    scratch_types=[
        pltpu.VMEM((120, E), jnp.float32),
        pltpu.SemaphoreType.DMA((_NBUF,)),
        pltpu.SemaphoreType.DMA((_NBUF,)),
    ],
)
def _broadcast_rows(table_hbm, out_hbm, buf, rsem, wsem):
    wid = lax.axis_index("s") * _NC + lax.axis_index("c")
    base = wid * _ROWS_PER_W

    def read_copy(c):
        k = c % _NBUF
        return pltpu.make_async_copy(
            table_hbm.at[pl.ds(base + _OFFS[c], _SIZES[c])],
            buf.at[pl.ds(_BOFF[k], _SIZES[c])],
            rsem.at[k],
        )

    def write_copy(c, b):
        k = c % _NBUF
        return pltpu.make_async_copy(
            buf.at[pl.ds(_BOFF[k], _SIZES[c])],
            out_hbm.at[b, pl.ds(base + _OFFS[c], _SIZES[c])],
            wsem.at[k],
        )

    for c in range(min(_NBUF - 1, _NCHUNK)):
        read_copy(c).start()
    for c in range(_NCHUNK):
        read_copy(c).wait()
        nxt = c + _NBUF - 1
        if nxt < _NCHUNK:
            if c >= 1:
                for b in range(B):
                    write_copy(c - 1, b).wait()
            read_copy(nxt).start()
        for b in range(B):
            write_copy(c, b).start()
    for c in range(max(0, _NCHUNK - _NBUF), _NCHUNK):
        for b in range(B):
            write_copy(c, b).wait()


def kernel(x, table):
    del x
    return _broadcast_rows(table)
